# Initial kernel scaffold; baseline (speedup 1.0000x reference)
import jax
import jax.numpy as jnp
from jax.experimental import pallas as pl


def kernel(x, weight, bias, gamma, beta):
    raise NotImplementedError("write your pallas kernel here")


# speedup vs baseline: 1.7941x; 1.7941x over previous
import jax
import jax.numpy as jnp
from jax import lax
from jax.experimental import pallas as pl
from jax.experimental.pallas import tpu as pltpu

_KS = 4
_ST = 2
_PD = 1
_EPS = 1e-5


def _ceil_to(a, b):
    return (a + b - 1) // b * b


def kernel(x, weight, bias, gamma, beta):
    del bias
    N, C, H, W = x.shape
    Cout = weight.shape[0]
    Hout = (H + 2 * _PD - _KS) // _ST + 1
    Wout = (W + 2 * _PD - _KS) // _ST + 1
    Hc, Wc = Hout + 1, Wout + 1
    M = Hout * Wc
    Mc = Hout * Wout
    C4 = 4 * C
    Mp = _ceil_to(Hc * Wc + 2, 16)
    count = N * Mc

    xp = jnp.pad(x, ((0, 0), (0, 0), (_PD, _PD), (_PD, _PD)))
    xs = xp.reshape(N, C, Hc, 2, Wc, 2).transpose(0, 2, 4, 3, 5, 1)
    xs = xs.reshape(N, Hc * Wc, C4).astype(jnp.bfloat16)
    xs = jnp.pad(xs, ((0, 0), (0, Mp - Hc * Wc), (0, 0)))

    wt = weight.reshape(Cout, C, 2, 2, 2, 2)
    wt = wt.transpose(3, 5, 1, 2, 4, 0)
    wt = wt.reshape(C4, 4 * Cout).astype(jnp.bfloat16)

    offs = tuple(dh * Wc + dw for dh in range(2) for dw in range(2))

    def conv_kernel(xs_ref, w_ref, y_ref, sum_ref, ssq_ref):
        t = jnp.dot(xs_ref[0], w_ref[...], preferred_element_type=jnp.float32)
        y = (t[offs[0]:offs[0] + M, :Cout]
             + t[offs[1]:offs[1] + M, Cout:2 * Cout]
             + t[offs[2]:offs[2] + M, 2 * Cout:3 * Cout]
             + t[offs[3]:offs[3] + M, 3 * Cout:])
        yc = jnp.concatenate(
            [y[i * Wc:i * Wc + Wout] for i in range(Hout)], axis=0)
        sum_ref[0] = jnp.sum(yc, axis=0, keepdims=True)
        ssq_ref[0] = jnp.sum(yc * yc, axis=0, keepdims=True)
        y_ref[0] = yc.astype(jnp.bfloat16)

    y, sums, ssqs = pl.pallas_call(
        conv_kernel,
        out_shape=(jax.ShapeDtypeStruct((N, Mc, Cout), jnp.bfloat16),
                   jax.ShapeDtypeStruct((N, 1, Cout), jnp.float32),
                   jax.ShapeDtypeStruct((N, 1, Cout), jnp.float32)),
        grid=(N,),
        in_specs=[pl.BlockSpec((1, Mp, C4), lambda n: (n, 0, 0)),
                  pl.BlockSpec((C4, 4 * Cout), lambda n: (0, 0))],
        out_specs=(pl.BlockSpec((1, Mc, Cout), lambda n: (n, 0, 0)),
                   pl.BlockSpec((1, 1, Cout), lambda n: (n, 0, 0)),
                   pl.BlockSpec((1, 1, Cout), lambda n: (n, 0, 0))),
        compiler_params=pltpu.CompilerParams(
            dimension_semantics=("parallel",)),
    )(xs, wt)

    total = jnp.sum(sums[:, 0, :], axis=0)
    total_sq = jnp.sum(ssqs[:, 0, :], axis=0)
    mean = total / jnp.float32(count)
    var = jnp.maximum(total_sq / jnp.float32(count) - mean * mean, 0.0)
    inv_std = lax.rsqrt(var + _EPS)
    scale = (gamma * inv_std).reshape(1, Cout)
    shift = (beta - mean * gamma * inv_std).reshape(1, Cout)

    def norm_kernel(y_ref, scale_ref, shift_ref, out_ref):
        z = jnp.maximum(y_ref[0].astype(jnp.float32) * scale_ref[...]
                        + shift_ref[...], 0.0)
        out_ref[0] = z.T

    out = pl.pallas_call(
        norm_kernel,
        out_shape=jax.ShapeDtypeStruct((N, Cout, Mc), jnp.float32),
        grid=(N,),
        in_specs=[pl.BlockSpec((1, Mc, Cout), lambda n: (n, 0, 0)),
                  pl.BlockSpec((1, Cout), lambda n: (0, 0)),
                  pl.BlockSpec((1, Cout), lambda n: (0, 0))],
        out_specs=pl.BlockSpec((1, Cout, Mc), lambda n: (n, 0, 0)),
        compiler_params=pltpu.CompilerParams(
            dimension_semantics=("parallel",)),
    )(y, scale, shift)

    return out.reshape(N, Cout, Hout, Wout)


# speedup vs baseline: 1.8002x; 1.0034x over previous
import jax
import jax.numpy as jnp
from jax import lax
from jax.experimental import pallas as pl
from jax.experimental.pallas import tpu as pltpu

_KS = 4
_ST = 2
_PD = 1
_EPS = 1e-5


def _ceil_to(a, b):
    return (a + b - 1) // b * b


def kernel(x, weight, bias, gamma, beta):
    del bias
    N, C, H, W = x.shape
    Cout = weight.shape[0]
    Hout = (H + 2 * _PD - _KS) // _ST + 1
    Wout = (W + 2 * _PD - _KS) // _ST + 1
    Hc, Wc = Hout + 1, Wout + 1
    M = Hout * Wc
    Mc = Hout * Wout
    C4 = 4 * C
    Mp = _ceil_to(Hc * Wc + 2, 16)
    count = N * Mc

    xp = jnp.pad(x, ((0, 0), (0, 0), (_PD, _PD), (_PD, _PD)))
    xs = xp.reshape(N, C, Hc, 2, Wc, 2).transpose(0, 2, 4, 3, 5, 1)
    xs = xs.reshape(N, Hc * Wc, C4).astype(jnp.bfloat16)
    xs = jnp.pad(xs, ((0, 0), (0, Mp - Hc * Wc), (0, 0)))

    wt = weight.reshape(Cout, C, 2, 2, 2, 2)
    wt = wt.transpose(3, 5, 1, 2, 4, 0)
    wt = wt.reshape(C4, 4 * Cout).astype(jnp.bfloat16)

    offs = tuple(dh * Wc + dw for dh in range(2) for dw in range(2))

    def conv_kernel(xs_ref, w_ref, y_ref, sum_ref, ssq_ref):
        t = jnp.dot(xs_ref[0], w_ref[...], preferred_element_type=jnp.float32)
        y = (t[offs[0]:offs[0] + M, :Cout]
             + t[offs[1]:offs[1] + M, Cout:2 * Cout]
             + t[offs[2]:offs[2] + M, 2 * Cout:3 * Cout]
             + t[offs[3]:offs[3] + M, 3 * Cout:])
        yc = jnp.concatenate(
            [y[i * Wc:i * Wc + Wout] for i in range(Hout)], axis=0)
        sum_ref[0] = jnp.sum(yc, axis=0, keepdims=True)
        ssq_ref[0] = jnp.sum(yc * yc, axis=0, keepdims=True)
        y_ref[0] = yc.astype(jnp.bfloat16)

    y, sums, ssqs = pl.pallas_call(
        conv_kernel,
        out_shape=(jax.ShapeDtypeStruct((N, Mc, Cout), jnp.bfloat16),
                   jax.ShapeDtypeStruct((N, 1, Cout), jnp.float32),
                   jax.ShapeDtypeStruct((N, 1, Cout), jnp.float32)),
        grid=(N,),
        in_specs=[pl.BlockSpec((1, Mp, C4), lambda n: (n, 0, 0)),
                  pl.BlockSpec((C4, 4 * Cout), lambda n: (0, 0))],
        out_specs=(pl.BlockSpec((1, Mc, Cout), lambda n: (n, 0, 0)),
                   pl.BlockSpec((1, 1, Cout), lambda n: (n, 0, 0)),
                   pl.BlockSpec((1, 1, Cout), lambda n: (n, 0, 0))),
        compiler_params=pltpu.CompilerParams(
            dimension_semantics=("parallel",)),
    )(xs, wt)

    gamma2 = gamma.reshape(1, Cout)
    beta2 = beta.reshape(1, Cout)
    inv_count = float(1.0 / count)

    def norm_kernel(y_ref, sums_ref, ssqs_ref, gamma_ref, beta_ref, out_ref):
        mean = jnp.sum(sums_ref[:, 0, :], axis=0, keepdims=True) * inv_count
        msq = jnp.sum(ssqs_ref[:, 0, :], axis=0, keepdims=True) * inv_count
        var = jnp.maximum(msq - mean * mean, 0.0)
        scale = gamma_ref[...] * lax.rsqrt(var + _EPS)
        shift = beta_ref[...] - mean * scale
        z = jnp.maximum(y_ref[0].astype(jnp.float32) * scale + shift, 0.0)
        out_ref[0] = z.T

    out = pl.pallas_call(
        norm_kernel,
        out_shape=jax.ShapeDtypeStruct((N, Cout, Mc), jnp.float32),
        grid=(N,),
        in_specs=[pl.BlockSpec((1, Mc, Cout), lambda n: (n, 0, 0)),
                  pl.BlockSpec((N, 1, Cout), lambda n: (0, 0, 0)),
                  pl.BlockSpec((N, 1, Cout), lambda n: (0, 0, 0)),
                  pl.BlockSpec((1, Cout), lambda n: (0, 0)),
                  pl.BlockSpec((1, Cout), lambda n: (0, 0))],
        out_specs=pl.BlockSpec((1, Cout, Mc), lambda n: (n, 0, 0)),
        compiler_params=pltpu.CompilerParams(
            dimension_semantics=("parallel",)),
    )(y, sums, ssqs, gamma2, beta2)

    return out.reshape(N, Cout, Hout, Wout)
